# Initial kernel scaffold; baseline (speedup 1.0000x reference)
import jax
import jax.numpy as jnp
from jax.experimental import pallas as pl


def kernel(x_seq, edge_index, W, b, Wih, Whh, bih, bhh, fc_w, fc_b):
    raise NotImplementedError("write your pallas kernel here")


# speedup vs baseline: 62.9203x; 62.9203x over previous
import functools

import jax
import jax.numpy as jnp
from jax import lax
from jax.experimental import pallas as pl
from jax.experimental.pallas import tpu as pltpu
from jax.experimental.pallas import tpu_sc as plsc

NC = 2
NS = 16
CB = 128
PADROWS = 256


def _make_deg_dis_kernel(T, NPAD, SLICE, CPT):
    t_per_core = T // NC
    mesh = plsc.VectorSubcoreMesh(core_axis_name="c", subcore_axis_name="s")

    @functools.partial(
        pl.kernel,
        out_type=jax.ShapeDtypeStruct((T * NPAD,), jnp.float32),
        mesh=mesh,
        compiler_params=pltpu.CompilerParams(use_tc_tiling_on_sc=False),
        scratch_types=[
            pltpu.VMEM((CPT, CB), jnp.int32),
            pltpu.VMEM((CB,), jnp.float32),
            pltpu.VMEM((SLICE,), jnp.float32),
            pltpu.VMEM_SHARED((NPAD,), jnp.float32),
        ],
    )
    def deg_kernel(edges_hbm, ones_hbm, zeros_hbm, deg_hbm, idx_v, ones_v,
                   work_v, deg_sh):
        cid = lax.axis_index("c")
        sid = lax.axis_index("s")
        start_c = sid * CPT
        my_slice = sid * SLICE
        pltpu.sync_copy(ones_hbm, ones_v)
        for tt in range(t_per_core):
            t = tt * NC + cid
            pltpu.sync_copy(zeros_hbm, work_v)
            pltpu.sync_copy(work_v, deg_sh.at[pl.ds(my_slice, SLICE)])
            pltpu.sync_copy(edges_hbm.at[t, 1, pl.ds(start_c, CPT), :], idx_v)
            plsc.subcore_barrier()

            def count_body(j, carry):
                pltpu.sync_copy(ones_v, deg_sh.at[idx_v.at[j]], add=True)
                return carry

            lax.fori_loop(0, CPT, count_body, 0)
            plsc.subcore_barrier()
            pltpu.sync_copy(deg_sh.at[pl.ds(my_slice, SLICE)],
                            deg_hbm.at[pl.ds(t * NPAD + my_slice, SLICE)])
            plsc.subcore_barrier()

    return deg_kernel


def _make_edge_kernel(T, N, HG, NROWS, ROWS_A, ROWS_B, CPT):
    t_per_core = T // NC
    mesh = plsc.VectorSubcoreMesh(core_axis_name="c", subcore_axis_name="s")

    @functools.partial(
        pl.kernel,
        out_type=jax.ShapeDtypeStruct((T, N, HG), jnp.float32),
        mesh=mesh,
        compiler_params=pltpu.CompilerParams(use_tc_tiling_on_sc=False),
        scratch_types=[
            pltpu.VMEM((CPT, CB), jnp.int32),
            pltpu.VMEM((CPT, CB), jnp.int32),
            pltpu.VMEM((CB, HG), jnp.float32),
            pltpu.VMEM((ROWS_A, HG), jnp.float32),
            pltpu.VMEM_SHARED((NROWS, HG), jnp.float32),
            pltpu.VMEM_SHARED((NROWS, HG), jnp.float32),
        ],
    )
    def edge_kernel(edges_hbm, hs_hbm, zeros_hbm, acc_hbm, src_v, dst_v,
                    rows_v, zero_v, hs_sh, acc_sh):
        cid = lax.axis_index("c")
        sid = lax.axis_index("s")
        start_c = sid * CPT
        pltpu.sync_copy(zeros_hbm, zero_v)
        for tt in range(t_per_core):
            t = tt * NC + cid
            @pl.when(sid < NS - 1)
            def _():
                r0 = sid * ROWS_A
                pltpu.sync_copy(zero_v, acc_sh.at[pl.ds(r0, ROWS_A), :])
                pltpu.sync_copy(hs_hbm.at[t, pl.ds(r0, ROWS_A), :],
                                hs_sh.at[pl.ds(r0, ROWS_A), :])

            @pl.when(sid == NS - 1)
            def _():
                r0 = (NS - 1) * ROWS_A
                pltpu.sync_copy(zero_v.at[pl.ds(0, ROWS_B), :],
                                acc_sh.at[pl.ds(r0, ROWS_B), :])
                pltpu.sync_copy(hs_hbm.at[t, pl.ds(r0, ROWS_B), :],
                                hs_sh.at[pl.ds(r0, ROWS_B), :])
                pltpu.sync_copy(zero_v.at[pl.ds(0, PADROWS), :],
                                hs_sh.at[pl.ds(N, PADROWS), :])
                pltpu.sync_copy(zero_v.at[pl.ds(0, PADROWS), :],
                                acc_sh.at[pl.ds(N, PADROWS), :])

            pltpu.sync_copy(edges_hbm.at[t, 0, pl.ds(start_c, CPT), :], src_v)
            pltpu.sync_copy(edges_hbm.at[t, 1, pl.ds(start_c, CPT), :], dst_v)
            plsc.subcore_barrier()

            def chunk_body(j, carry):
                pltpu.sync_copy(hs_sh.at[src_v.at[j]], rows_v)
                pltpu.sync_copy(rows_v, acc_sh.at[dst_v.at[j]], add=True)
                return carry

            lax.fori_loop(0, CPT, chunk_body, 0)
            plsc.subcore_barrier()

            @pl.when(sid < NS - 1)
            def _():
                r0 = sid * ROWS_A
                pltpu.sync_copy(acc_sh.at[pl.ds(r0, ROWS_A), :],
                                acc_hbm.at[t, pl.ds(r0, ROWS_A), :])

            @pl.when(sid == NS - 1)
            def _():
                r0 = (NS - 1) * ROWS_A
                pltpu.sync_copy(acc_sh.at[pl.ds(r0, ROWS_B), :],
                                acc_hbm.at[t, pl.ds(r0, ROWS_B), :])

            plsc.subcore_barrier()

    return edge_kernel


def _hs_body(x_ref, w_ref, deg_ref, hs_ref, dis_ref):
    dis = lax.rsqrt(deg_ref[0] + 1.0)
    h = jnp.dot(x_ref[0], w_ref[...], preferred_element_type=jnp.float32)
    hs_ref[0] = h * dis
    dis_ref[0] = dis


def _make_final_body(T, N, HG, HL):
    def body(acc_ref, hs_ref, dis_ref, b_ref, wih_ref, whh_ref,
             bih_ref, bhh_ref, fcw_ref, fcb_ref, out_ref, seq_scr):
        t = pl.program_id(0)
        agg = (acc_ref[0] + hs_ref[0]) * dis_ref[0] + b_ref[...]
        seq_t = jnp.mean(jax.nn.relu(agg), axis=0)
        seq_scr[pl.ds(t, 1), :] = seq_t[None, :]

        @pl.when(t == T - 1)
        def _():
            h = jnp.zeros((1, HL), jnp.float32)
            c = jnp.zeros((1, HL), jnp.float32)
            for ti in range(T):
                xt = seq_scr[pl.ds(ti, 1), :]
                g = (jnp.dot(xt, wih_ref[...],
                             preferred_element_type=jnp.float32,
                             precision=lax.Precision.HIGHEST)
                     + jnp.dot(h, whh_ref[...],
                               preferred_element_type=jnp.float32,
                               precision=lax.Precision.HIGHEST)
                     + bih_ref[...] + bhh_ref[...])
                gi = jax.nn.sigmoid(g[:, :HL])
                gf = jax.nn.sigmoid(g[:, HL:2 * HL])
                gg = jnp.tanh(g[:, 2 * HL:3 * HL])
                go = jax.nn.sigmoid(g[:, 3 * HL:])
                c = gf * c + gi * gg
                h = go * jnp.tanh(c)
            out_ref[...] = (
                jnp.dot(h, fcw_ref[...], preferred_element_type=jnp.float32,
                        precision=lax.Precision.HIGHEST)
                + fcb_ref[...])

    return body


def kernel(x_seq, edge_index, W, b, Wih, Whh, bih, bhh, fc_w, fc_b):
    T, N, FIN = x_seq.shape
    E = edge_index.shape[2]
    HG = W.shape[1]
    HL = Whh.shape[1]

    NROWS = N + PADROWS
    n_chunks = E // CB
    CPT = ((n_chunks + NS * 8 - 1) // (NS * 8)) * 8
    n_chunks_pad = CPT * NS
    SLICE = ((NROWS + NS * 16 - 1) // (NS * 16)) * 16
    NPAD = SLICE * NS
    ROWS_A = ((N + NS * 8 - 1) // (NS * 8)) * 8
    ROWS_B = N - (NS - 1) * ROWS_A

    edges4 = edge_index.reshape(T, 2, n_chunks, CB)
    n_dummy = (n_chunks_pad - n_chunks) * CB
    pad_idx = (N + (jnp.arange(n_dummy, dtype=jnp.int32) % PADROWS))
    edges_pad = jnp.broadcast_to(
        pad_idx.reshape(1, 1, n_chunks_pad - n_chunks, CB),
        (T, 2, n_chunks_pad - n_chunks, CB))
    edges4p = jnp.concatenate([edges4, edges_pad], axis=2)

    deg_kernel = _make_deg_dis_kernel(T, NPAD, SLICE, CPT)
    deg_flat = deg_kernel(edges4p, jnp.ones((CB,), jnp.float32),
                          jnp.zeros((SLICE,), jnp.float32))
    deg2 = deg_flat.reshape(T, NPAD)[:, :N, None]

    hs, dis2 = pl.pallas_call(
        _hs_body,
        grid=(T,),
        in_specs=[
            pl.BlockSpec((1, N, FIN), lambda t: (t, 0, 0)),
            pl.BlockSpec((FIN, HG), lambda t: (0, 0)),
            pl.BlockSpec((1, N, 1), lambda t: (t, 0, 0)),
        ],
        out_specs=[
            pl.BlockSpec((1, N, HG), lambda t: (t, 0, 0)),
            pl.BlockSpec((1, N, 1), lambda t: (t, 0, 0)),
        ],
        out_shape=[
            jax.ShapeDtypeStruct((T, N, HG), jnp.float32),
            jax.ShapeDtypeStruct((T, N, 1), jnp.float32),
        ],
    )(x_seq, W, deg2)

    edge_kernel = _make_edge_kernel(T, N, HG, NROWS, ROWS_A, ROWS_B, CPT)
    acc = edge_kernel(edges4p, hs, jnp.zeros((ROWS_A, HG), jnp.float32))

    full = lambda shape: pl.BlockSpec(shape, lambda t: tuple(0 for _ in shape))
    out2d = pl.pallas_call(
        _make_final_body(T, N, HG, HL),
        grid=(T,),
        in_specs=[
            pl.BlockSpec((1, N, HG), lambda t: (t, 0, 0)),
            pl.BlockSpec((1, N, HG), lambda t: (t, 0, 0)),
            pl.BlockSpec((1, N, 1), lambda t: (t, 0, 0)),
            full((1, HG)),
            full((HG, 4 * HL)), full((HL, 4 * HL)),
            full((1, 4 * HL)), full((1, 4 * HL)),
            full((HL, 1)), full((1, 1)),
        ],
        out_specs=pl.BlockSpec((1, 1), lambda t: (0, 0)),
        out_shape=jax.ShapeDtypeStruct((1, 1), jnp.float32),
        scratch_shapes=[pltpu.VMEM((T, HG), jnp.float32)],
    )(acc, hs, dis2, b.reshape(1, HG), Wih.T, Whh.T,
      bih.reshape(1, 4 * HL), bhh.reshape(1, 4 * HL),
      fc_w.T, fc_b.reshape(1, 1))
    return out2d.reshape(())
